# baseline (speedup 1.0000x reference)
import functools

import jax
import jax.numpy as jnp
from jax import lax
from jax.experimental import pallas as pl
from jax.experimental.pallas import tpu as pltpu
from jax.experimental.pallas import tpu_sc as plsc

N = 10000
E = 320000
D = 128
DH = D // 2
NC = 2
NS = 16
NW = NC * NS

DEG_EPW = E // NW
DEG_CH = 80
DEG_NCHUNK = DEG_EPW // DEG_CH

EPW = E // NS
CH = 80
NCHUNK = EPW // CH
NBUF = 8
LOOK = 4
STEADY = ((NCHUNK - 2 * LOOK) // NBUF) * NBUF

NPAD = 10240
RPT = NPAD // NS
BLK = 2048
GRID = 5

_mesh = plsc.VectorSubcoreMesh(core_axis_name="c", subcore_axis_name="s")
_sc_params = pltpu.CompilerParams(needs_layout_passes=False)
_sc_linear_params = pltpu.CompilerParams(needs_layout_passes=False,
                                         use_tc_tiling_on_sc=False)


@functools.partial(
    pl.kernel,
    out_type=jax.ShapeDtypeStruct((NW, NPAD), jnp.float32),
    mesh=_mesh,
    compiler_params=_sc_linear_params,
    scratch_types=[
        pltpu.VMEM((DEG_NCHUNK, DEG_CH), jnp.int32),
        pltpu.VMEM((NPAD,), jnp.float32),
    ],
)
def _deg_kernel(row_hbm, degp_hbm, idx_v, deg_v):
    cid = lax.axis_index("c")
    sid = lax.axis_index("s")
    w = sid * NC + cid
    pltpu.sync_copy(row_hbm.at[w // 2, pl.ds((w % 2) * DEG_NCHUNK, DEG_NCHUNK)],
                    idx_v)

    def zero_body(i, carry):
        deg_v[pl.ds(i * 16, 16)] = jnp.zeros((16,), jnp.float32)
        return carry

    lax.fori_loop(0, NPAD // 16, zero_body, 0)

    def chunk_body(c, carry):
        for j in range(DEG_CH // 16):
            idx = idx_v[c, pl.ds(j * 16, 16)]
            cnt, last = plsc.scan_count(idx)
            plsc.addupdate_scatter(deg_v, [idx], cnt.astype(jnp.float32), mask=last)
        return carry

    lax.fori_loop(0, DEG_NCHUNK, chunk_body, 0)
    pltpu.sync_copy(deg_v, degp_hbm.at[w])


@functools.partial(
    pl.kernel,
    out_type=jax.ShapeDtypeStruct((NC * NPAD, DH), jnp.float32),
    mesh=_mesh,
    compiler_params=_sc_linear_params,
    scratch_types=(
        [
            pltpu.VMEM((NCHUNK, CH), jnp.int32),
            pltpu.VMEM((NCHUNK, CH), jnp.int32),
            pltpu.VMEM_SHARED((NPAD, DH), jnp.float32),
        ]
        + [pltpu.VMEM((CH, DH), jnp.float32) for _ in range(NBUF)]
        + [pltpu.SemaphoreType.DMA for _ in range(2 * NBUF)]
    ),
)
def _agg_kernel(y_hbm, col_hbm, row_hbm, z_hbm, col_v, row_v, z_sh, *rest):
    buf = rest[:NBUF]
    sg = rest[NBUF:2 * NBUF]
    ss = rest[2 * NBUF:3 * NBUF]
    cid = lax.axis_index("c")
    sid = lax.axis_index("s")
    pltpu.sync_copy(col_hbm.at[sid], col_v)
    pltpu.sync_copy(row_hbm.at[sid], row_v)

    def col_off_body(c, carry):
        for j in range(CH // 16):
            v = col_v[c, pl.ds(j * 16, 16)]
            col_v[c, pl.ds(j * 16, 16)] = v * 2 + cid
        return carry

    lax.fori_loop(0, NCHUNK, col_off_body, 0)

    def zbuf_body(i, carry):
        for j in range(DH // 16):
            buf[0][i, pl.ds(j * 16, 16)] = jnp.zeros((16,), jnp.float32)
        return carry

    lax.fori_loop(0, CH, zbuf_body, 0)
    zbase = sid * RPT
    for k in range(RPT // CH):
        pltpu.sync_copy(buf[0], z_sh.at[pl.ds(zbase + k * CH, CH)])
    rem = RPT % CH
    if rem:
        pltpu.sync_copy(buf[0].at[pl.ds(0, rem)],
                        z_sh.at[pl.ds(zbase + (RPT // CH) * CH, rem)])
    plsc.subcore_barrier()

    def gather(c, k):
        pltpu.async_copy(y_hbm.at[col_v.at[c]], buf[k], sg[k])

    def wait_gather(k):
        pltpu.make_async_copy(y_hbm.at[col_v.at[0]], buf[k], sg[k]).wait()

    def scatter(c, k):
        pltpu.async_copy(buf[k], z_sh.at[row_v.at[c]], ss[k], add=True)

    def wait_scatter(k):
        pltpu.make_async_copy(buf[k], z_sh.at[row_v.at[0]], ss[k]).wait()

    for k in range(LOOK):
        gather(k, k)

    for c in range(LOOK):
        k = c % NBUF
        wait_gather(k)
        scatter(c, k)
        kk = (c + LOOK) % NBUF
        gather(c + LOOK, kk)

    def group_body(g, carry):
        base = LOOK + g * NBUF
        for j in range(NBUF):
            k = (LOOK + j) % NBUF
            wait_gather(k)
            scatter(base + j, k)
            kk = (LOOK + j + LOOK) % NBUF
            wait_scatter(kk)
            gather(base + j + LOOK, kk)
        return carry

    lax.fori_loop(0, STEADY // NBUF, group_body, 0)

    for c in range(LOOK + STEADY, NCHUNK - LOOK):
        k = c % NBUF
        wait_gather(k)
        scatter(c, k)
        kk = (c + LOOK) % NBUF
        wait_scatter(kk)
        gather(c + LOOK, kk)

    for c in range(NCHUNK - LOOK, NCHUNK):
        k = c % NBUF
        wait_gather(k)
        scatter(c, k)

    for c in range(NCHUNK - NBUF, NCHUNK):
        wait_scatter(c % NBUF)

    plsc.subcore_barrier()
    off = cid * NPAD + sid * RPT
    pltpu.sync_copy(z_sh.at[pl.ds(sid * RPT, RPT)], z_hbm.at[pl.ds(off, RPT)])


def _proj_body(x_ref, w_ref, degp_ref, y_ref):
    xp = lax.dot_general(x_ref[...], w_ref[...], (((1,), (1,)), ((), ())),
                         preferred_element_type=jnp.float32)
    deg = jnp.maximum(jnp.sum(degp_ref[...], axis=0), 1.0)
    y_ref[...] = xp * lax.rsqrt(deg)[:, None]


def _final_body(za_ref, zb_ref, degp_ref, b_ref, o_ref):
    deg = jnp.maximum(jnp.sum(degp_ref[...], axis=0), 1.0)
    s = lax.rsqrt(deg)[:, None]
    z = jnp.concatenate([za_ref[...], zb_ref[...]], axis=1)
    o_ref[...] = z * s + b_ref[...]


def kernel(x, edge_index, W, b):
    row16 = edge_index[0].reshape(NS, NCHUNK, CH)
    col16 = edge_index[1].reshape(NS, NCHUNK, CH)

    degp = _deg_kernel(row16)

    y = pl.pallas_call(
        _proj_body,
        grid=(GRID,),
        in_specs=[
            pl.BlockSpec((BLK, D), lambda i: (i, 0)),
            pl.BlockSpec((D, D), lambda i: (0, 0)),
            pl.BlockSpec((NW, BLK), lambda i: (0, i)),
        ],
        out_specs=pl.BlockSpec((BLK, D), lambda i: (i, 0)),
        out_shape=jax.ShapeDtypeStruct((N, D), jnp.float32),
    )(x, W, degp)

    zflat = _agg_kernel(y.reshape(2 * N, DH), col16, row16)

    out = pl.pallas_call(
        _final_body,
        grid=(GRID,),
        in_specs=[
            pl.BlockSpec((BLK, DH), lambda i: (i, 0)),
            pl.BlockSpec((BLK, DH), lambda i: (NPAD // BLK + i, 0)),
            pl.BlockSpec((NW, BLK), lambda i: (0, i)),
            pl.BlockSpec((1, D), lambda i: (0, 0)),
        ],
        out_specs=pl.BlockSpec((BLK, D), lambda i: (i, 0)),
        out_shape=jax.ShapeDtypeStruct((N, D), jnp.float32),
    )(zflat, zflat, degp, b.reshape(1, D))

    return out


# speedup vs baseline: 1.0641x; 1.0641x over previous
import functools

import jax
import jax.numpy as jnp
from jax import lax
from jax.experimental import pallas as pl
from jax.experimental.pallas import tpu as pltpu
from jax.experimental.pallas import tpu_sc as plsc

N = 10000
E = 320000
D = 128
DH = D // 2
NC = 2
NS = 16
NW = NC * NS

DEG_EPW = E // NW
DEG_CH = 80
DEG_NCHUNK = DEG_EPW // DEG_CH

EPW = E // NS
CH = 80
NCHUNK = EPW // CH
NBUF = 8
LOOK = 4
STEADY = ((NCHUNK - 2 * LOOK) // NBUF) * NBUF

NPAD = 10240
RPT = NPAD // NS
BLK = 2048
GRID = 5

_mesh = plsc.VectorSubcoreMesh(core_axis_name="c", subcore_axis_name="s")
_sc_params = pltpu.CompilerParams(needs_layout_passes=False)
_sc_linear_params = pltpu.CompilerParams(needs_layout_passes=False,
                                         use_tc_tiling_on_sc=False)


@functools.partial(
    pl.kernel,
    out_type=jax.ShapeDtypeStruct((NW, NPAD), jnp.float32),
    mesh=_mesh,
    compiler_params=_sc_linear_params,
    scratch_types=[
        pltpu.VMEM((DEG_NCHUNK, DEG_CH), jnp.int32),
        pltpu.VMEM((NPAD,), jnp.float32),
    ],
)
def _deg_kernel(row_hbm, degp_hbm, idx_v, deg_v):
    cid = lax.axis_index("c")
    sid = lax.axis_index("s")
    w = sid * NC + cid
    pltpu.sync_copy(row_hbm.at[w // 2, pl.ds((w % 2) * DEG_NCHUNK, DEG_NCHUNK)],
                    idx_v)

    def zero_body(i, carry):
        deg_v[pl.ds(i * 16, 16)] = jnp.zeros((16,), jnp.float32)
        return carry

    lax.fori_loop(0, NPAD // 16, zero_body, 0)

    def chunk_body(c, carry):
        for j in range(DEG_CH // 16):
            idx = idx_v[c, pl.ds(j * 16, 16)]
            cnt, last = plsc.scan_count(idx)
            plsc.addupdate_scatter(deg_v, [idx], cnt.astype(jnp.float32), mask=last)
        return carry

    lax.fori_loop(0, DEG_NCHUNK, chunk_body, 0)
    pltpu.sync_copy(deg_v, degp_hbm.at[w])


@functools.partial(
    pl.kernel,
    out_type=jax.ShapeDtypeStruct((NPAD, D), jnp.float32),
    mesh=_mesh,
    compiler_params=_sc_linear_params,
    scratch_types=(
        [
            pltpu.VMEM((NCHUNK, CH), jnp.int32),
            pltpu.VMEM((NCHUNK, CH), jnp.int32),
            pltpu.VMEM_SHARED((NPAD, DH), jnp.float32),
        ]
        + [pltpu.VMEM((CH, DH), jnp.float32) for _ in range(NBUF)]
        + [pltpu.SemaphoreType.DMA for _ in range(2 * NBUF)]
    ),
)
def _agg_kernel(y_hbm, col_hbm, row_hbm, z_hbm, col_v, row_v, z_sh, *rest):
    buf = rest[:NBUF]
    sg = rest[NBUF:2 * NBUF]
    ss = rest[2 * NBUF:3 * NBUF]
    cid = lax.axis_index("c")
    sid = lax.axis_index("s")
    pltpu.sync_copy(col_hbm.at[sid], col_v)
    pltpu.sync_copy(row_hbm.at[sid], row_v)

    def col_off_body(c, carry):
        for j in range(CH // 16):
            v = col_v[c, pl.ds(j * 16, 16)]
            col_v[c, pl.ds(j * 16, 16)] = v * 2 + cid
        return carry

    lax.fori_loop(0, NCHUNK, col_off_body, 0)

    def zbuf_body(i, carry):
        for j in range(DH // 16):
            buf[0][i, pl.ds(j * 16, 16)] = jnp.zeros((16,), jnp.float32)
        return carry

    lax.fori_loop(0, CH, zbuf_body, 0)
    zbase = sid * RPT
    for k in range(RPT // CH):
        pltpu.sync_copy(buf[0], z_sh.at[pl.ds(zbase + k * CH, CH)])
    rem = RPT % CH
    if rem:
        pltpu.sync_copy(buf[0].at[pl.ds(0, rem)],
                        z_sh.at[pl.ds(zbase + (RPT // CH) * CH, rem)])
    plsc.subcore_barrier()

    def gather(c, k):
        pltpu.async_copy(y_hbm.at[col_v.at[c]], buf[k], sg[k])

    def wait_gather(k):
        pltpu.make_async_copy(y_hbm.at[col_v.at[0]], buf[k], sg[k]).wait()

    def scatter(c, k):
        pltpu.async_copy(buf[k], z_sh.at[row_v.at[c]], ss[k], add=True)

    def wait_scatter(k):
        pltpu.make_async_copy(buf[k], z_sh.at[row_v.at[0]], ss[k]).wait()

    for k in range(LOOK):
        gather(k, k)

    for c in range(LOOK):
        k = c % NBUF
        wait_gather(k)
        scatter(c, k)
        kk = (c + LOOK) % NBUF
        gather(c + LOOK, kk)

    def group_body(g, carry):
        base = LOOK + g * NBUF
        for j in range(NBUF):
            k = (LOOK + j) % NBUF
            wait_gather(k)
            scatter(base + j, k)
            kk = (LOOK + j + LOOK) % NBUF
            wait_scatter(kk)
            gather(base + j + LOOK, kk)
        return carry

    lax.fori_loop(0, STEADY // NBUF, group_body, 0)

    for c in range(LOOK + STEADY, NCHUNK - LOOK):
        k = c % NBUF
        wait_gather(k)
        scatter(c, k)
        kk = (c + LOOK) % NBUF
        wait_scatter(kk)
        gather(c + LOOK, kk)

    for c in range(NCHUNK - LOOK, NCHUNK):
        k = c % NBUF
        wait_gather(k)
        scatter(c, k)

    for c in range(NCHUNK - NBUF, NCHUNK):
        wait_scatter(c % NBUF)

    plsc.subcore_barrier()
    pltpu.sync_copy(z_sh.at[pl.ds(sid * RPT, RPT)],
                    z_hbm.at[pl.ds(sid * RPT, RPT), pl.ds(cid * DH, DH)])


def _proj_body(x_ref, w_ref, degp_ref, y_ref):
    xp = lax.dot_general(x_ref[...], w_ref[...], (((1,), (1,)), ((), ())),
                         preferred_element_type=jnp.float32)
    deg = jnp.maximum(jnp.sum(degp_ref[...], axis=0), 1.0)
    y_ref[...] = xp * lax.rsqrt(deg)[:, None]


def _final_body(z_ref, degp_ref, b_ref, o_ref):
    deg = jnp.maximum(jnp.sum(degp_ref[...], axis=0), 1.0)
    s = lax.rsqrt(deg)[:, None]
    o_ref[...] = z_ref[...] * s + b_ref[...]


def kernel(x, edge_index, W, b):
    row16 = edge_index[0].reshape(NS, NCHUNK, CH)
    col16 = edge_index[1].reshape(NS, NCHUNK, CH)

    degp = _deg_kernel(row16)

    y = pl.pallas_call(
        _proj_body,
        grid=(GRID,),
        in_specs=[
            pl.BlockSpec((BLK, D), lambda i: (i, 0)),
            pl.BlockSpec((D, D), lambda i: (0, 0)),
            pl.BlockSpec((NW, BLK), lambda i: (0, i)),
        ],
        out_specs=pl.BlockSpec((BLK, D), lambda i: (i, 0)),
        out_shape=jax.ShapeDtypeStruct((N, D), jnp.float32),
    )(x, W, degp)

    z = _agg_kernel(y.reshape(2 * N, DH), col16, row16)

    out = pl.pallas_call(
        _final_body,
        grid=(GRID,),
        in_specs=[
            pl.BlockSpec((BLK, D), lambda i: (i, 0)),
            pl.BlockSpec((NW, BLK), lambda i: (0, i)),
            pl.BlockSpec((1, D), lambda i: (0, 0)),
        ],
        out_specs=pl.BlockSpec((BLK, D), lambda i: (i, 0)),
        out_shape=jax.ShapeDtypeStruct((N, D), jnp.float32),
    )(z, degp, b.reshape(1, D))

    return out
